# baseline (speedup 1.0000x reference)
import jax
import jax.numpy as jnp
from jax.experimental import pallas as pl


def _body(f_ref, m_ref, t_ref, o_ref):
    R = f_ref.shape[0]
    m = m_ref[0].astype(jnp.int32).reshape(R, 1) != 0
    o_ref[...] = jnp.where(m, t_ref[...], f_ref[...])


def kernel(features, mask, mask_token):
    B, S, D = features.shape
    N = B * S
    R = 512
    f2 = features.reshape(N, D)
    m3 = mask.reshape(N // R, 1, R)
    t2 = mask_token.reshape(1, D)
    grid = (N // R,)
    out = pl.pallas_call(
        _body,
        grid=grid,
        in_specs=[
            pl.BlockSpec((R, D), lambda i: (i, 0)),
            pl.BlockSpec((1, 1, R), lambda i: (i, 0, 0)),
            pl.BlockSpec((1, D), lambda i: (0, 0)),
        ],
        out_specs=pl.BlockSpec((R, D), lambda i: (i, 0)),
        out_shape=jax.ShapeDtypeStruct((N, D), features.dtype),
    )(f2, m3, t2)
    return out.reshape(B, S, D)


# speedup vs baseline: 1.1239x; 1.1239x over previous
import jax
import jax.numpy as jnp
from jax.experimental import pallas as pl


def _body(f_ref, m_ref, t_ref, o_ref):
    R = f_ref.shape[0]
    m = m_ref[0].astype(jnp.int32).reshape(R, 1) != 0
    o_ref[...] = jnp.where(m, t_ref[...], f_ref[...])


def kernel(features, mask, mask_token):
    B, S, D = features.shape
    N = B * S
    R = 2048
    f2 = features.reshape(N, D)
    m3 = mask.reshape(N // R, 1, R)
    t2 = mask_token.reshape(1, D)
    grid = (N // R,)
    out = pl.pallas_call(
        _body,
        grid=grid,
        in_specs=[
            pl.BlockSpec((R, D), lambda i: (i, 0)),
            pl.BlockSpec((1, 1, R), lambda i: (i, 0, 0)),
            pl.BlockSpec((1, D), lambda i: (0, 0)),
        ],
        out_specs=pl.BlockSpec((R, D), lambda i: (i, 0)),
        out_shape=jax.ShapeDtypeStruct((N, D), features.dtype),
    )(f2, m3, t2)
    return out.reshape(B, S, D)
